# Initial kernel scaffold; baseline (speedup 1.0000x reference)
import jax
import jax.numpy as jnp
from jax.experimental import pallas as pl


def kernel(story, kb_len, conv_len, hidden, dh_outputs, adj, C0, C1, C2, C3):
    raise NotImplementedError("write your pallas kernel here")


# speedup vs baseline: 5.7878x; 5.7878x over previous
import functools

import jax
import jax.numpy as jnp
from jax import lax
from jax.experimental import pallas as pl
from jax.experimental.pallas import tpu as pltpu
from jax.experimental.pallas import tpu_sc as plsc

_NC = 2
_NS = 16
_NW = _NC * _NS
_IDXW = 128


def _sc_embed_grid(story3d, T01, T23, *, n_rows, S, D2):
    per_w = n_rows // _NW
    chunk = 160
    n_chunks = per_w // chunk
    idx_rows_w = per_w * S // _IDXW
    idx_rows_c = chunk * S // _IDXW

    mesh = plsc.VectorSubcoreMesh(core_axis_name="core",
                                  subcore_axis_name="subcore")
    out_sds = tuple(
        jax.ShapeDtypeStruct((_NW * n_chunks, chunk, D2), jnp.float32)
        for _ in range(2))

    @functools.partial(
        pl.kernel,
        out_type=out_sds,
        mesh=mesh,
        scratch_types=[
            pltpu.VMEM((idx_rows_w, _IDXW), jnp.int32),
            pltpu.VMEM((chunk * S, D2), jnp.float32),
            pltpu.VMEM((chunk, D2), jnp.float32),
            pltpu.SemaphoreType.DMA,
        ],
    )
    def k(story_hbm, t01, t23, o01, o23, idx_v, rows_v, out_v, sem):
        tables = [t01, t23]
        outs = [o01, o23]
        wid = lax.axis_index("subcore") * _NC + lax.axis_index("core")
        pltpu.sync_copy(story_hbm.at[wid], idx_v)

        @pl.loop(0, n_chunks)
        def _(c):
            for h in range(2):
                for j in range(idx_rows_c):
                    pltpu.async_copy(
                        tables[h].at[idx_v.at[c * idx_rows_c + j]],
                        rows_v.at[pl.ds(j * _IDXW, _IDXW)],
                        sem,
                    )
                for j in range(idx_rows_c):
                    pltpu.make_async_copy(
                        tables[h].at[idx_v.at[c * idx_rows_c + j]],
                        rows_v.at[pl.ds(j * _IDXW, _IDXW)],
                        sem,
                    ).wait()

                @pl.loop(0, chunk)
                def _(i):
                    r = i * S
                    for dc in range(D2 // 16):
                        sl = pl.ds(dc * 16, 16)
                        acc = rows_v[r, sl]
                        for s in range(1, S):
                            acc = acc + rows_v[r + s, sl]
                        out_v[i, sl] = acc

                pltpu.sync_copy(out_v, outs[h].at[wid * n_chunks + c])

    return k(story3d, T01, T23)


def _tc_hops_body(kb_ref, cl_ref, p01_ref, p23_ref, dh_ref, u0_ref,
                  prob_ref, uout_ref, *, M, D, CONV):
    b = pl.program_id(0)
    kb = kb_ref[b]
    cl = cl_ref[b]
    dh = dh_ref[0]

    iota_m = lax.broadcasted_iota(jnp.int32, (M, CONV), 0)
    iota_j = lax.broadcasted_iota(jnp.int32, (M, CONV), 1)
    sel = ((iota_m - kb) == iota_j) & (iota_j < cl)
    L = jnp.dot(sel.astype(jnp.float32), dh,
                preferred_element_type=jnp.float32,
                precision=lax.Precision.HIGHEST)
    L2 = jnp.concatenate([L, L], axis=1)

    A01 = p01_ref[0] + L2
    A23 = p23_ref[0] + L2

    u = u0_ref[0]
    zero = jnp.zeros((1, D), jnp.float32)

    def hop(A_logit, z, A_out):
        logit = jnp.sum(A_logit * z, axis=1)
        p = jax.nn.softmax(logit)
        w = jnp.sum(A_out * p[:, None], axis=0, keepdims=True)
        return logit, w

    _, w = hop(A01, jnp.concatenate([u, zero], axis=1), A01)
    u = u + w[:, D:]
    _, w = hop(A01, jnp.concatenate([zero, u], axis=1), A23)
    u = u + w[:, :D]
    logit, w = hop(A23, jnp.concatenate([u, zero], axis=1), A23)
    u = u + w[:, D:]

    prob_ref[0, 0] = jax.nn.sigmoid(logit)
    uout_ref[0] = u


def _tc_hops(kb_len, conv_len, P01, P23, dh, u0, *, interpret=False):
    B, M, D2 = P01.shape
    D = D2 // 2
    CONV = dh.shape[1]
    p_spec = pl.BlockSpec((1, M, D2), lambda b: (b, 0, 0))
    return pl.pallas_call(
        functools.partial(_tc_hops_body, M=M, D=D, CONV=CONV),
        grid=(B,),
        in_specs=[
            pl.BlockSpec(memory_space=pltpu.SMEM),
            pl.BlockSpec(memory_space=pltpu.SMEM),
            p_spec, p_spec,
            pl.BlockSpec((1, CONV, D), lambda b: (b, 0, 0)),
            pl.BlockSpec((1, 1, D), lambda b: (b, 0, 0)),
        ],
        out_specs=[
            pl.BlockSpec((1, 1, M), lambda b: (b, 0, 0)),
            pl.BlockSpec((1, 1, D), lambda b: (b, 0, 0)),
        ],
        out_shape=[
            jax.ShapeDtypeStruct((B, 1, M), jnp.float32),
            jax.ShapeDtypeStruct((B, 1, D), jnp.float32),
        ],
        interpret=interpret,
    )(kb_len, conv_len, P01, P23, dh, u0.reshape(B, 1, D))


def kernel(story, kb_len, conv_len, hidden, dh_outputs, adj, C0, C1, C2, C3):
    B, M, S = story.shape
    D = C0.shape[1]
    n_rows = B * M
    T01 = jnp.concatenate([C0, C1], axis=1)
    T23 = jnp.concatenate([C2, C3], axis=1)
    story3d = story.reshape(_NW, n_rows * S // (_NW * _IDXW), _IDXW)
    P01, P23 = _sc_embed_grid(story3d, T01, T23,
                              n_rows=n_rows, S=S, D2=2 * D)
    prob, u = _tc_hops(kb_len, conv_len,
                       P01.reshape(B, M, 2 * D), P23.reshape(B, M, 2 * D),
                       dh_outputs, hidden[0])
    return prob.reshape(B, M), u.reshape(B, D)


# speedup vs baseline: 8.1893x; 1.4149x over previous
import functools

import jax
import jax.numpy as jnp
from jax import lax
from jax.experimental import pallas as pl
from jax.experimental.pallas import tpu as pltpu
from jax.experimental.pallas import tpu_sc as plsc

_NC = 2
_NS = 16
_NW = _NC * _NS
_IDXW = 128


def _sc_embed_grid(story3d, T01, T23, *, n_rows, S, D2):
    per_w = n_rows // _NW
    chunk = 160
    n_chunks = per_w // chunk
    idx_rows_c = chunk * S // _IDXW
    idx_rows_w = per_w * S // _IDXW

    mesh = plsc.VectorSubcoreMesh(core_axis_name="core",
                                  subcore_axis_name="subcore")
    out_sds = tuple(
        jax.ShapeDtypeStruct((n_rows, D2), jnp.float32) for _ in range(2))

    @functools.partial(
        pl.kernel,
        out_type=out_sds,
        mesh=mesh,
        scratch_types=[
            pltpu.VMEM((idx_rows_w, _IDXW), jnp.int32),
            pltpu.VMEM((chunk * S, D2), jnp.float32),
            pltpu.VMEM((chunk, D2), jnp.float32),
            pltpu.SemaphoreType.DMA,
        ],
    )
    def k(story_hbm, t01, t23, o01, o23, idx_v, rows_v, out_v, sem):
        tables = [t01, t23]
        outs = [o01, o23]
        wid = lax.axis_index("subcore") * _NC + lax.axis_index("core")
        pltpu.sync_copy(story_hbm.at[wid], idx_v)
        out_base = wid * per_w

        @pl.loop(0, n_chunks)
        def _(c):
            for h in range(2):
                for j in range(idx_rows_c):
                    pltpu.async_copy(
                        tables[h].at[idx_v.at[c * idx_rows_c + j]],
                        rows_v.at[pl.ds(j * _IDXW, _IDXW)],
                        sem,
                    )
                for j in range(idx_rows_c):
                    pltpu.make_async_copy(
                        tables[h].at[idx_v.at[c * idx_rows_c + j]],
                        rows_v.at[pl.ds(j * _IDXW, _IDXW)],
                        sem,
                    ).wait()

                @pl.loop(0, chunk)
                def _(i):
                    r = i * S
                    for dc in range(D2 // 16):
                        sl = pl.ds(dc * 16, 16)
                        acc = rows_v[r, sl]
                        for s in range(1, S):
                            acc = acc + rows_v[r + s, sl]
                        out_v[i, sl] = acc

                pltpu.sync_copy(
                    out_v, outs[h].at[pl.ds(out_base + c * chunk, chunk)])

    return k(story3d, T01, T23)


def _tc_hops_body(kb_ref, cl_ref, p01_ref, p23_ref, dh_ref, u0_ref,
                  prob_ref, uout_ref, *, BB, M, D, CONV):
    kb = kb_ref[...][:, 0]
    cl = cl_ref[...][:, 0]
    dh = dh_ref[...]

    iota_m = lax.broadcasted_iota(jnp.int32, (BB, M, CONV), 1)
    iota_j = lax.broadcasted_iota(jnp.int32, (BB, M, CONV), 2)
    sel = ((iota_m - kb[:, None, None]) == iota_j) & \
          (iota_j < cl[:, None, None])
    L = lax.dot_general(sel.astype(jnp.float32), dh,
                        dimension_numbers=(((2,), (1,)), ((0,), (0,))),
                        preferred_element_type=jnp.float32,
                        precision=lax.Precision.HIGHEST)
    L2 = jnp.concatenate([L, L], axis=2)

    A01 = p01_ref[...].reshape(BB, M, 2 * D) + L2
    A23 = p23_ref[...].reshape(BB, M, 2 * D) + L2

    u = u0_ref[...][:, 0, :]
    zero = jnp.zeros((BB, D), jnp.float32)

    def hop(A_logit, z, A_out):
        logit = jnp.sum(A_logit * z[:, None, :], axis=2)
        p = jax.nn.softmax(logit, axis=1)
        w = jnp.sum(A_out * p[:, :, None], axis=1)
        return logit, w

    _, w = hop(A01, jnp.concatenate([u, zero], axis=1), A01)
    u = u + w[:, D:]
    _, w = hop(A01, jnp.concatenate([zero, u], axis=1), A23)
    u = u + w[:, :D]
    logit, w = hop(A23, jnp.concatenate([u, zero], axis=1), A23)
    u = u + w[:, D:]

    prob_ref[:, 0, :] = jax.nn.sigmoid(logit)
    uout_ref[:, 0, :] = u


def _tc_hops(kb_len, conv_len, P01, P23, dh, u0, *, interpret=False):
    B, CONV, D = dh.shape
    M = P01.shape[0] // B
    D2 = P01.shape[1]
    BB = 8
    return pl.pallas_call(
        functools.partial(_tc_hops_body, BB=BB, M=M, D=D, CONV=CONV),
        grid=(B // BB,),
        in_specs=[
            pl.BlockSpec((BB, 1), lambda b: (b, 0)),
            pl.BlockSpec((BB, 1), lambda b: (b, 0)),
            pl.BlockSpec((BB * M, D2), lambda b: (b, 0)),
            pl.BlockSpec((BB * M, D2), lambda b: (b, 0)),
            pl.BlockSpec((BB, CONV, D), lambda b: (b, 0, 0)),
            pl.BlockSpec((BB, 1, D), lambda b: (b, 0, 0)),
        ],
        out_specs=[
            pl.BlockSpec((BB, 1, M), lambda b: (b, 0, 0)),
            pl.BlockSpec((BB, 1, D), lambda b: (b, 0, 0)),
        ],
        out_shape=[
            jax.ShapeDtypeStruct((B, 1, M), jnp.float32),
            jax.ShapeDtypeStruct((B, 1, D), jnp.float32),
        ],
        interpret=interpret,
    )(kb_len.reshape(B, 1), conv_len.reshape(B, 1), P01, P23, dh,
      u0.reshape(B, 1, D))


def kernel(story, kb_len, conv_len, hidden, dh_outputs, adj, C0, C1, C2, C3):
    B, M, S = story.shape
    D = C0.shape[1]
    n_rows = B * M
    T01 = jnp.concatenate([C0, C1], axis=1)
    T23 = jnp.concatenate([C2, C3], axis=1)
    story3d = story.reshape(_NW, n_rows * S // (_NW * _IDXW), _IDXW)
    P01, P23 = _sc_embed_grid(story3d, T01, T23,
                              n_rows=n_rows, S=S, D2=2 * D)
    prob, u = _tc_hops(kb_len, conv_len, P01, P23, dh_outputs, hidden[0])
    return prob.reshape(B, M), u.reshape(B, D)


# speedup vs baseline: 9.4809x; 1.1577x over previous
import functools

import jax
import jax.numpy as jnp
from jax import lax
from jax.experimental import pallas as pl
from jax.experimental.pallas import tpu as pltpu
from jax.experimental.pallas import tpu_sc as plsc

_NC = 2
_NS = 16
_NW = _NC * _NS
_IDXW = 128


def _sc_embed_grid(story3d, T01, T23, *, n_rows, S, D2):
    per_w = n_rows // _NW
    chunk = 64
    n_chunks = per_w // chunk
    idx_rows_c = chunk * S // _IDXW
    idx_rows_w = per_w * S // _IDXW

    mesh = plsc.VectorSubcoreMesh(core_axis_name="core",
                                  subcore_axis_name="subcore")
    out_sds = tuple(
        jax.ShapeDtypeStruct((n_rows, D2), jnp.float32) for _ in range(2))

    @functools.partial(
        pl.kernel,
        out_type=out_sds,
        mesh=mesh,
        scratch_types=[
            pltpu.VMEM((idx_rows_w, _IDXW), jnp.int32),
            pltpu.VMEM((chunk * S, D2), jnp.float32),
            pltpu.VMEM((chunk * S, D2), jnp.float32),
            pltpu.VMEM((chunk, D2), jnp.float32),
            pltpu.SemaphoreType.DMA,
            pltpu.SemaphoreType.DMA,
        ],
    )
    def k(story_hbm, t01, t23, o01, o23, idx_v, rows0, rows1, out_v,
          sem0, sem1):
        tables = [t01, t23]
        outs = [o01, o23]
        bufs = [rows0, rows1]
        sems = [sem0, sem1]
        wid = lax.axis_index("subcore") * _NC + lax.axis_index("core")
        pltpu.sync_copy(story_hbm.at[wid], idx_v)
        out_base = wid * per_w

        def enqueue(h, c):
            for j in range(idx_rows_c):
                pltpu.async_copy(
                    tables[h].at[idx_v.at[c * idx_rows_c + j]],
                    bufs[h].at[pl.ds(j * _IDXW, _IDXW)],
                    sems[h],
                )

        def drain(h, c):
            for j in range(idx_rows_c):
                pltpu.make_async_copy(
                    tables[h].at[idx_v.at[c * idx_rows_c + j]],
                    bufs[h].at[pl.ds(j * _IDXW, _IDXW)],
                    sems[h],
                ).wait()

        enqueue(0, 0)

        @pl.loop(0, n_chunks)
        def _(c):
            for h in range(2):
                if h == 0:
                    enqueue(1, c)
                else:
                    @pl.when(c + 1 < n_chunks)
                    def _():
                        enqueue(0, c + 1)
                drain(h, c)

                @pl.loop(0, chunk)
                def _(i):
                    r = i * S
                    for dc in range(D2 // 16):
                        sl = pl.ds(dc * 16, 16)
                        acc = bufs[h][r, sl]
                        for s in range(1, S):
                            acc = acc + bufs[h][r + s, sl]
                        out_v[i, sl] = acc

                pltpu.sync_copy(
                    out_v, outs[h].at[pl.ds(out_base + c * chunk, chunk)])

    return k(story3d, T01, T23)


def _tc_hops_body(kb_ref, cl_ref, p01_ref, p23_ref, dh_ref, u0_ref,
                  prob_ref, uout_ref, *, BB, M, D, CONV):
    kb = kb_ref[...][:, 0]
    cl = cl_ref[...][:, 0]
    dh = dh_ref[...]

    iota_m = lax.broadcasted_iota(jnp.int32, (BB, M, CONV), 1)
    iota_j = lax.broadcasted_iota(jnp.int32, (BB, M, CONV), 2)
    sel = ((iota_m - kb[:, None, None]) == iota_j) & \
          (iota_j < cl[:, None, None])
    L = lax.dot_general(sel.astype(jnp.float32), dh,
                        dimension_numbers=(((2,), (1,)), ((0,), (0,))),
                        preferred_element_type=jnp.float32,
                        precision=lax.Precision.HIGHEST)
    L2 = jnp.concatenate([L, L], axis=2)

    A01 = p01_ref[...].reshape(BB, M, 2 * D) + L2
    A23 = p23_ref[...].reshape(BB, M, 2 * D) + L2

    u = u0_ref[...]
    zero = jnp.zeros((BB, D), jnp.float32)

    def hop(A_logit, z, A_out):
        logit = jnp.sum(A_logit * z[:, None, :], axis=2)
        p = jax.nn.softmax(logit, axis=1)
        w = jnp.sum(A_out * p[:, :, None], axis=1)
        return logit, w

    _, w = hop(A01, jnp.concatenate([u, zero], axis=1), A01)
    u = u + w[:, D:]
    _, w = hop(A01, jnp.concatenate([zero, u], axis=1), A23)
    u = u + w[:, :D]
    logit, w = hop(A23, jnp.concatenate([u, zero], axis=1), A23)
    u = u + w[:, D:]

    prob_ref[...] = jax.nn.sigmoid(logit)
    uout_ref[...] = u


def _tc_hops(kb_len, conv_len, P01, P23, dh, u0, *, interpret=False):
    B, CONV, D = dh.shape
    M = P01.shape[0] // B
    D2 = P01.shape[1]
    BB = 8
    return pl.pallas_call(
        functools.partial(_tc_hops_body, BB=BB, M=M, D=D, CONV=CONV),
        grid=(B // BB,),
        in_specs=[
            pl.BlockSpec((BB, 1), lambda b: (b, 0)),
            pl.BlockSpec((BB, 1), lambda b: (b, 0)),
            pl.BlockSpec((BB * M, D2), lambda b: (b, 0)),
            pl.BlockSpec((BB * M, D2), lambda b: (b, 0)),
            pl.BlockSpec((BB, CONV, D), lambda b: (b, 0, 0)),
            pl.BlockSpec((BB, D), lambda b: (b, 0)),
        ],
        out_specs=[
            pl.BlockSpec((BB, M), lambda b: (b, 0)),
            pl.BlockSpec((BB, D), lambda b: (b, 0)),
        ],
        out_shape=[
            jax.ShapeDtypeStruct((B, M), jnp.float32),
            jax.ShapeDtypeStruct((B, D), jnp.float32),
        ],
        interpret=interpret,
    )(kb_len.reshape(B, 1), conv_len.reshape(B, 1), P01, P23, dh, u0)


def kernel(story, kb_len, conv_len, hidden, dh_outputs, adj, C0, C1, C2, C3):
    B, M, S = story.shape
    D = C0.shape[1]
    n_rows = B * M
    T01 = jnp.concatenate([C0, C1], axis=1)
    T23 = jnp.concatenate([C2, C3], axis=1)
    story3d = story.reshape(_NW, n_rows * S // (_NW * _IDXW), _IDXW)
    P01, P23 = _sc_embed_grid(story3d, T01, T23,
                              n_rows=n_rows, S=S, D2=2 * D)
    prob, u = _tc_hops(kb_len, conv_len, P01, P23, dh_outputs, hidden[0])
    return prob, u


# speedup vs baseline: 11.6526x; 1.2291x over previous
import functools

import jax
import jax.numpy as jnp
from jax import lax
from jax.experimental import pallas as pl
from jax.experimental.pallas import tpu as pltpu
from jax.experimental.pallas import tpu_sc as plsc

_NC = 2
_NS = 16
_NW = _NC * _NS
_IDXW = 128


def _sc_embed_grid(story3d, T01, T23, *, n_rows, S, D2):
    per_w = n_rows // _NW
    chunk = 64
    n_chunks = per_w // chunk
    idx_rows_c = chunk * S // _IDXW
    idx_rows_w = per_w * S // _IDXW

    mesh = plsc.VectorSubcoreMesh(core_axis_name="core",
                                  subcore_axis_name="subcore")
    out_sds = tuple(
        jax.ShapeDtypeStruct((n_rows, D2), jnp.float32) for _ in range(2))

    @functools.partial(
        pl.kernel,
        out_type=out_sds,
        mesh=mesh,
        scratch_types=[
            pltpu.VMEM((idx_rows_w, _IDXW), jnp.int32),
            pltpu.VMEM((chunk * S, D2), jnp.float32),
            pltpu.VMEM((chunk * S, D2), jnp.float32),
            pltpu.VMEM((chunk, D2), jnp.float32),
            pltpu.SemaphoreType.DMA,
            pltpu.SemaphoreType.DMA,
        ],
    )
    def k(story_hbm, t01, t23, o01, o23, idx_v, rows0, rows1, out_v,
          sem0, sem1):
        tables = [t01, t23]
        outs = [o01, o23]
        bufs = [rows0, rows1]
        sems = [sem0, sem1]
        wid = lax.axis_index("subcore") * _NC + lax.axis_index("core")
        pltpu.sync_copy(story_hbm.at[wid], idx_v)
        out_base = wid * per_w

        def enqueue(h, c):
            for j in range(idx_rows_c):
                pltpu.async_copy(
                    tables[h].at[idx_v.at[c * idx_rows_c + j]],
                    bufs[h].at[pl.ds(j * _IDXW, _IDXW)],
                    sems[h],
                )

        def drain(h, c):
            for j in range(idx_rows_c):
                pltpu.make_async_copy(
                    tables[h].at[idx_v.at[c * idx_rows_c + j]],
                    bufs[h].at[pl.ds(j * _IDXW, _IDXW)],
                    sems[h],
                ).wait()

        enqueue(0, 0)

        @pl.loop(0, n_chunks)
        def _(c):
            for h in range(2):
                if h == 0:
                    enqueue(1, c)
                else:
                    @pl.when(c + 1 < n_chunks)
                    def _():
                        enqueue(0, c + 1)
                drain(h, c)

                @plsc.parallel_loop(0, chunk, 1, unroll=4)
                def _(i):
                    r = i * S
                    for dc in range(D2 // 16):
                        sl = pl.ds(dc * 16, 16)
                        acc = bufs[h][r, sl]
                        for s in range(1, S):
                            acc = acc + bufs[h][r + s, sl]
                        out_v[i, sl] = acc

                pltpu.sync_copy(
                    out_v, outs[h].at[pl.ds(out_base + c * chunk, chunk)])

    return k(story3d, T01, T23)


def _tc_hops_body(kb_ref, cl_ref, p01_ref, p23_ref, dh_ref, u0_ref,
                  prob_ref, uout_ref, *, BB, M, D, CONV):
    kb = kb_ref[...][:, 0]
    cl = cl_ref[...][:, 0]
    dh = dh_ref[...]

    iota_m = lax.broadcasted_iota(jnp.int32, (BB, M, CONV), 1)
    iota_j = lax.broadcasted_iota(jnp.int32, (BB, M, CONV), 2)
    sel = ((iota_m - kb[:, None, None]) == iota_j) & \
          (iota_j < cl[:, None, None])
    L = lax.dot_general(sel.astype(jnp.float32), dh,
                        dimension_numbers=(((2,), (1,)), ((0,), (0,))),
                        preferred_element_type=jnp.float32,
                        precision=lax.Precision.HIGHEST)
    L2 = jnp.concatenate([L, L], axis=2)

    A01 = p01_ref[...].reshape(BB, M, 2 * D) + L2
    A23 = p23_ref[...].reshape(BB, M, 2 * D) + L2

    u = u0_ref[...]
    zero = jnp.zeros((BB, D), jnp.float32)

    def hop(A_logit, z, A_out):
        logit = jnp.sum(A_logit * z[:, None, :], axis=2)
        p = jax.nn.softmax(logit, axis=1)
        w = jnp.sum(A_out * p[:, :, None], axis=1)
        return logit, w

    _, w = hop(A01, jnp.concatenate([u, zero], axis=1), A01)
    u = u + w[:, D:]
    _, w = hop(A01, jnp.concatenate([zero, u], axis=1), A23)
    u = u + w[:, :D]
    logit, w = hop(A23, jnp.concatenate([u, zero], axis=1), A23)
    u = u + w[:, D:]

    prob_ref[...] = jax.nn.sigmoid(logit)
    uout_ref[...] = u


def _tc_hops(kb_len, conv_len, P01, P23, dh, u0, *, interpret=False):
    B, CONV, D = dh.shape
    M = P01.shape[0] // B
    D2 = P01.shape[1]
    BB = 8
    return pl.pallas_call(
        functools.partial(_tc_hops_body, BB=BB, M=M, D=D, CONV=CONV),
        grid=(B // BB,),
        in_specs=[
            pl.BlockSpec((BB, 1), lambda b: (b, 0)),
            pl.BlockSpec((BB, 1), lambda b: (b, 0)),
            pl.BlockSpec((BB * M, D2), lambda b: (b, 0)),
            pl.BlockSpec((BB * M, D2), lambda b: (b, 0)),
            pl.BlockSpec((BB, CONV, D), lambda b: (b, 0, 0)),
            pl.BlockSpec((BB, D), lambda b: (b, 0)),
        ],
        out_specs=[
            pl.BlockSpec((BB, M), lambda b: (b, 0)),
            pl.BlockSpec((BB, D), lambda b: (b, 0)),
        ],
        out_shape=[
            jax.ShapeDtypeStruct((B, M), jnp.float32),
            jax.ShapeDtypeStruct((B, D), jnp.float32),
        ],
        interpret=interpret,
    )(kb_len.reshape(B, 1), conv_len.reshape(B, 1), P01, P23, dh, u0)


def kernel(story, kb_len, conv_len, hidden, dh_outputs, adj, C0, C1, C2, C3):
    B, M, S = story.shape
    D = C0.shape[1]
    n_rows = B * M
    T01 = jnp.concatenate([C0, C1], axis=1)
    T23 = jnp.concatenate([C2, C3], axis=1)
    story3d = story.reshape(_NW, n_rows * S // (_NW * _IDXW), _IDXW)
    P01, P23 = _sc_embed_grid(story3d, T01, T23,
                              n_rows=n_rows, S=S, D2=2 * D)
    prob, u = _tc_hops(kb_len, conv_len, P01, P23, dh_outputs, hidden[0])
    return prob, u
